# baseline (speedup 1.0000x reference)
import functools

import jax
import jax.numpy as jnp
from jax import lax
from jax.experimental import pallas as pl
from jax.experimental.pallas import tpu as pltpu
from jax.experimental.pallas import tpu_sc as plsc

N_PAPER = 100000
N_AUTHOR = 50000
D = 64

NC = 2
NS = 16
L = 16

CHUNK = 25000
ROW_STRIPE = 1563
ACC_ROWS = NS * ROW_STRIPE
CNT_STRIPE = 1568
CNT_STRIDE = NS * CNT_STRIPE

B = 2048
G = 128
GQ = G // L
SENT = 2 ** 30


def _spmm_body(ncpc, nb, with_counts, refs):
    (x_hbm, src_hbm, dst_hbm, agg_out, cnt_out,
     stage_src, stage_dst, ssel, dsel, rows, zrow, zcnt, ones,
     acc, cacc, gsem, ssem, csem) = refs

    c = lax.axis_index("c")
    s = lax.axis_index("s")
    base_e = s * (nb * B)

    def init_zrow(i, _):
        for q in range(4):
            zrow[i, pl.ds(q * L, L)] = jnp.zeros((L,), jnp.float32)
        return 0
    lax.fori_loop(0, 64, init_zrow, 0)
    for q in range(4):
        zcnt[pl.ds(q * L, L)] = jnp.zeros((L,), jnp.float32)
    ones[...] = jnp.ones((L,), jnp.float32)

    for ci in range(ncpc):
        chunk = c * ncpc + ci
        lo = chunk * CHUNK
        r0 = s * ROW_STRIPE
        c0 = s * CNT_STRIPE

        for q in range(24):
            pltpu.sync_copy(zrow, acc.at[pl.ds(r0 + q * 64, 64)])
        pltpu.sync_copy(zrow.at[pl.ds(0, 27)],
                        acc.at[pl.ds(r0 + 24 * 64, 27)])
        if with_counts:
            for q in range(24):
                pltpu.sync_copy(zcnt, cacc.at[pl.ds(c0 + q * 64, 64)])
            pltpu.sync_copy(zcnt.at[pl.ds(0, 32)],
                            cacc.at[pl.ds(c0 + 24 * 64, 32)])
        plsc.subcore_barrier()

        def block(j, carry):
            off = base_e + j * B
            pltpu.sync_copy(src_hbm.at[pl.ds(off, B)], stage_src)
            pltpu.sync_copy(dst_hbm.at[pl.ds(off, B)], stage_dst)

            def subb(t, carry2):
                o = t * G
                for q in range(GQ):
                    d = stage_dst[pl.ds(o + q * L, L)]
                    sv = stage_src[pl.ds(o + q * L, L)]
                    d_rel = d - lo
                    m = (d_rel >= 0) & (d_rel < CHUNK)
                    dsel[pl.ds(q * L, L)] = jnp.where(
                        m, d_rel, jnp.full((L,), CHUNK, jnp.int32))
                    ssel[pl.ds(q * L, L)] = jnp.where(
                        m, sv, jnp.zeros((L,), jnp.int32))
                pltpu.async_copy(x_hbm.at[ssel], rows, gsem).wait()
                descs = []
                for q in range(GQ):
                    dv = dsel[pl.ds(q * L, L)]
                    descs.append(pltpu.async_copy(
                        rows.at[pl.ds(q * L, L)], acc.at[dv], ssem, add=True))
                    if with_counts:
                        descs.append(pltpu.async_copy(
                            ones, cacc.at[dv], csem, add=True))
                for dsc in descs:
                    dsc.wait()
                return carry2

            lax.fori_loop(0, B // G, subb, jnp.int32(0))
            return carry

        lax.fori_loop(0, nb, block, jnp.int32(0))

        plsc.subcore_barrier()
        pltpu.sync_copy(acc.at[pl.ds(r0, ROW_STRIPE)],
                        agg_out.at[chunk, pl.ds(r0, ROW_STRIPE)])
        if with_counts:
            pltpu.sync_copy(
                cacc.at[pl.ds(c0, CNT_STRIPE)],
                cnt_out.at[pl.ds(chunk * CNT_STRIDE + c0, CNT_STRIPE)])
        plsc.subcore_barrier()


def _spmm(x, src, dst, n_dst, with_counts):
    e = src.shape[0]
    eb = e // NS
    assert eb * NS == e
    nb = -(-eb // B)
    pad = nb * B - eb
    sw = src.reshape(NS, eb)
    dw = dst.reshape(NS, eb)
    if pad:
        sw = jnp.concatenate([sw, jnp.zeros((NS, pad), jnp.int32)], axis=1)
        dw = jnp.concatenate(
            [dw, jnp.full((NS, pad), SENT, jnp.int32)], axis=1)
    src2 = sw.reshape(-1)
    dst2 = dw.reshape(-1)
    nchunks = n_dst // CHUNK
    ncpc = nchunks // NC

    out_type = [jax.ShapeDtypeStruct((nchunks, ACC_ROWS, D), jnp.float32),
                jax.ShapeDtypeStruct((nchunks * CNT_STRIDE,), jnp.float32)]

    mesh = plsc.VectorSubcoreMesh(core_axis_name="c", subcore_axis_name="s")
    body = functools.partial(_spmm_body, ncpc, nb, with_counts)
    fn = pl.kernel(
        lambda *refs: body(refs),
        out_type=out_type,
        mesh=mesh,
        compiler_params=pltpu.CompilerParams(use_tc_tiling_on_sc=False),
        scratch_types=[
            pltpu.VMEM((B,), jnp.int32),
            pltpu.VMEM((B,), jnp.int32),
            pltpu.VMEM((G,), jnp.int32),
            pltpu.VMEM((G,), jnp.int32),
            pltpu.VMEM((G, D), jnp.float32),
            pltpu.VMEM((64, D), jnp.float32),
            pltpu.VMEM((64,), jnp.float32),
            pltpu.VMEM((L,), jnp.float32),
            pltpu.VMEM_SHARED((ACC_ROWS, D), jnp.float32),
            pltpu.VMEM_SHARED((CNT_STRIDE,), jnp.float32),
            pltpu.SemaphoreType.DMA,
            pltpu.SemaphoreType.DMA,
            pltpu.SemaphoreType.DMA,
        ],
        name=f"spmm_{n_dst}_{e}_{int(with_counts)}",
    )
    agg4, cnt1 = fn(x, src2, dst2)
    agg = agg4[:, :CHUNK].reshape(nchunks * CHUNK, D)
    if with_counts:
        cnt = cnt1.reshape(nchunks, CNT_STRIDE)[:, :CHUNK].reshape(-1)
        return agg, cnt
    return agg


BLK = 2000


def _fused_body(relu, nrel, *refs):
    it = iter(refs)
    acc = None
    for _ in range(nrel):
        a = next(it)[...]
        cn = next(it)[...]
        wl = next(it)[...]
        mean = a / jnp.maximum(cn, 1.0)
        t = jnp.dot(mean, wl, preferred_element_type=jnp.float32)
        acc = t if acc is None else acc + t
    x = next(it)[...]
    wr = next(it)[...]
    b = next(it)[...]
    out = next(it)
    acc = acc + jnp.dot(x, wr, preferred_element_type=jnp.float32) + b
    out[...] = jnp.maximum(acc, 0.0) if relu else acc


def _fused_sage(aggs, cnts, wls, x, wr, b, relu):
    n = x.shape[0]
    nrel = len(aggs)
    grid = (n // BLK,)
    row_spec = pl.BlockSpec((BLK, D), lambda i: (i, 0))
    cnt_spec = pl.BlockSpec((BLK, 1), lambda i: (i, 0))
    w_spec = pl.BlockSpec((D, D), lambda i: (0, 0))
    b_spec = pl.BlockSpec((1, D), lambda i: (0, 0))
    in_specs = []
    args = []
    for a, cn, wl in zip(aggs, cnts, wls):
        in_specs += [row_spec, cnt_spec, w_spec]
        args += [a, cn.reshape(-1, 1), wl]
    in_specs += [row_spec, w_spec, b_spec]
    args += [x, wr, b.reshape(1, D)]
    return pl.pallas_call(
        functools.partial(_fused_body, relu, nrel),
        grid=grid,
        in_specs=in_specs,
        out_specs=row_spec,
        out_shape=jax.ShapeDtypeStruct((n, D), jnp.float32),
    )(*args)


def kernel(x_paper, x_author, ei_cites, ei_writes, ei_rev_writes,
           W1c_l, W1c_r, b1c, W1w_l, W1w_r, b1w, W1r_l, W1r_r, b1r,
           W2c_l, W2c_r, b2c, W2w_l, W2w_r, b2w, W2r_l, W2r_r, b2r):
    sc, dc = ei_cites[0], ei_cites[1]
    sw, dw = ei_writes[0], ei_writes[1]
    sr, dr = ei_rev_writes[0], ei_rev_writes[1]

    agg1c, cntc = _spmm(x_paper, sc, dc, N_PAPER, True)
    agg1w, cntw = _spmm(x_author, sw, dw, N_PAPER, True)
    agg1r, cntr = _spmm(x_paper, sr, dr, N_AUTHOR, True)

    p1 = _fused_sage([agg1c, agg1w], [cntc, cntw], [W1c_l, W1w_l],
                     x_paper, W1c_r + W1w_r, b1c + b1w, relu=True)
    a1 = _fused_sage([agg1r], [cntr], [W1r_l],
                     x_author, W1r_r, b1r, relu=True)

    agg2c = _spmm(p1, sc, dc, N_PAPER, False)
    agg2w = _spmm(a1, sw, dw, N_PAPER, False)

    p2 = _fused_sage([agg2c, agg2w], [cntc, cntw], [W2c_l, W2w_l],
                     p1, W2c_r + W2w_r, b2c + b2w, relu=False)
    return p2
